# baseline (speedup 1.0000x reference)
import functools

import jax
import jax.numpy as jnp
from jax import lax
from jax.experimental import pallas as pl
from jax.experimental.pallas import tpu as pltpu
from jax.experimental.pallas import tpu_sc as plsc

DIM = 64
SCALE = 8.0

_NC = 2
_NS = 16
_LANES = 16
_NW = _NC * _NS

_C = 256


@functools.partial(jax.jit, static_argnums=(2, 3))
def _lookup(table128, x_t, l_seq, b_total):
    stripe = b_total // _NW
    n_chunk = l_seq * (stripe // _C)
    mesh = plsc.VectorSubcoreMesh(core_axis_name="c", subcore_axis_name="s")

    @functools.partial(
        pl.kernel,
        mesh=mesh,
        out_type=jax.ShapeDtypeStruct((l_seq, DIM, b_total), jnp.float32),
        scratch_types=[
            [pltpu.VMEM((_C,), jnp.int32) for _ in range(2)],
            [pltpu.VMEM((_C, 128), jnp.float32) for _ in range(2)],
            [pltpu.VMEM((DIM, _C), jnp.float32) for _ in range(2)],
            [pltpu.SemaphoreType.DMA for _ in range(2)],
            [pltpu.SemaphoreType.DMA for _ in range(2)],
        ],
        compiler_params=pltpu.CompilerParams(needs_layout_passes=False),
    )
    def k(tab_hbm, xt_hbm, out_hbm, idx_v, pairs_v, tv_v, gsem, ssem):
        wid = lax.axis_index("s") * _NC + lax.axis_index("c")
        b0w = wid * stripe
        per_l = stripe // _C

        def chunk_lb(t):
            l = t // per_l
            b0 = b0w + (t % per_l) * _C
            return l, b0

        def start_gather(t, buf):
            l, b0 = chunk_lb(t)
            pltpu.sync_copy(xt_hbm.at[l, pl.ds(b0, _C)], idx_v[buf])
            return pltpu.async_copy(tab_hbm.at[idx_v[buf]], pairs_v[buf], gsem[buf])

        start_gather(0, 0)

        iota16 = lax.iota(jnp.int32, _LANES)

        def outer(tt, carry):
            for b in (0, 1):
                t = 2 * tt + b
                nxt = 1 - b

                if b == 0:
                    start_gather(t + 1, nxt)
                else:
                    @pl.when(tt < (n_chunk // 2 - 1))
                    def _():
                        start_gather(t + 1, nxt)

                pltpu.make_async_copy(
                    tab_hbm.at[idx_v[b]], pairs_v[b], gsem[b]
                ).wait()

                @pl.when(tt >= 1)
                def _():
                    l2, b02 = chunk_lb(t - 2)
                    pltpu.make_async_copy(
                        tv_v[b], out_hbm.at[l2, :, pl.ds(b02, _C)], ssem[b]
                    ).wait()

                def trans_body(ib, c2):
                    i_vec = ib * _LANES + iota16
                    for d in range(DIM):
                        d_vec = jnp.full((_LANES,), d, jnp.int32)
                        vals = plsc.load_gather(pairs_v[b], [i_vec, d_vec])
                        tv_v[b][d, pl.ds(ib * _LANES, _LANES)] = vals * SCALE
                    return c2

                lax.fori_loop(0, _C // _LANES, trans_body, 0)

                l, b0 = chunk_lb(t)
                pltpu.async_copy(
                    tv_v[b], out_hbm.at[l, :, pl.ds(b0, _C)], ssem[b]
                )
            return carry

        lax.fori_loop(0, n_chunk // 2, outer, 0)

        for b in (0, 1):
            t = n_chunk - 2 + b
            l2, b02 = chunk_lb(t)
            pltpu.make_async_copy(
                tv_v[b], out_hbm.at[l2, :, pl.ds(b02, _C)], ssem[b]
            ).wait()

    return k(table128, x_t)


def kernel(x, table):
    b, l = x.shape
    x_t = jnp.transpose(x).astype(jnp.int32)
    table128 = jnp.pad(table, ((0, 0), (0, 128 - DIM)))
    res = _lookup(table128, x_t, l, b)
    return jnp.transpose(res, (2, 0, 1))


# speedup vs baseline: 1.1805x; 1.1805x over previous
import functools

import jax
import jax.numpy as jnp
from jax import lax
from jax.experimental import pallas as pl
from jax.experimental.pallas import tpu as pltpu
from jax.experimental.pallas import tpu_sc as plsc

DIM = 64
SCALE = 8.0

_NC = 2
_NS = 16
_LANES = 16
_NW = _NC * _NS

_C = 256


@functools.partial(jax.jit, static_argnums=(2, 3))
def _lookup(table128, x_t, l_seq, b_total):
    stripe = b_total // _NW
    n_chunk = l_seq * (stripe // _C)
    mesh = plsc.VectorSubcoreMesh(core_axis_name="c", subcore_axis_name="s")

    @functools.partial(
        pl.kernel,
        mesh=mesh,
        out_type=jax.ShapeDtypeStruct((l_seq, DIM, b_total), jnp.float32),
        scratch_types=[
            [pltpu.VMEM((_C,), jnp.int32) for _ in range(2)],
            [pltpu.VMEM((_C, 128), jnp.float32) for _ in range(2)],
            [pltpu.VMEM((DIM, _C), jnp.float32) for _ in range(2)],
            [pltpu.SemaphoreType.DMA for _ in range(2)],
            [pltpu.SemaphoreType.DMA for _ in range(2)],
        ],
        compiler_params=pltpu.CompilerParams(needs_layout_passes=False),
    )
    def k(tab_hbm, xt_hbm, out_hbm, idx_v, pairs_v, tv_v, gsem, ssem):
        wid = lax.axis_index("s") * _NC + lax.axis_index("c")
        b0w = wid * stripe
        per_l = stripe // _C

        def chunk_lb(t):
            l = t // per_l
            b0 = b0w + (t % per_l) * _C
            return l, b0

        def start_gather(t, buf):
            l, b0 = chunk_lb(t)
            pltpu.sync_copy(xt_hbm.at[l, pl.ds(b0, _C)], idx_v[buf])
            return pltpu.async_copy(tab_hbm.at[idx_v[buf]], pairs_v[buf], gsem[buf])

        start_gather(0, 0)

        iota16 = lax.iota(jnp.int32, _LANES)

        def outer(tt, carry):
            for b in (0, 1):
                t = 2 * tt + b
                nxt = 1 - b

                if b == 0:
                    start_gather(t + 1, nxt)
                else:
                    @pl.when(tt < (n_chunk // 2 - 1))
                    def _():
                        start_gather(t + 1, nxt)

                pltpu.make_async_copy(
                    tab_hbm.at[idx_v[b]], pairs_v[b], gsem[b]
                ).wait()

                @pl.when(tt >= 1)
                def _():
                    l2, b02 = chunk_lb(t - 2)
                    pltpu.make_async_copy(
                        tv_v[b], out_hbm.at[l2, :, pl.ds(b02, _C)], ssem[b]
                    ).wait()

                def trans_body(i4, c2):
                    for u in range(4):
                        i = i4 * 4 + u
                        i_splat = jnp.full((_LANES,), 0, jnp.int32) + i
                        for kk in range(DIM // _LANES):
                            vals = pairs_v[b][i, pl.ds(kk * _LANES, _LANES)]
                            plsc.store_scatter(
                                tv_v[b],
                                [kk * _LANES + iota16, i_splat],
                                vals * SCALE,
                            )
                    return c2

                lax.fori_loop(0, _C // 4, trans_body, 0)

                l, b0 = chunk_lb(t)
                pltpu.async_copy(
                    tv_v[b], out_hbm.at[l, :, pl.ds(b0, _C)], ssem[b]
                )
            return carry

        lax.fori_loop(0, n_chunk // 2, outer, 0)

        for b in (0, 1):
            t = n_chunk - 2 + b
            l2, b02 = chunk_lb(t)
            pltpu.make_async_copy(
                tv_v[b], out_hbm.at[l2, :, pl.ds(b02, _C)], ssem[b]
            ).wait()

    return k(table128, x_t)


def kernel(x, table):
    b, l = x.shape
    x_t = jnp.transpose(x).astype(jnp.int32)
    table128 = jnp.pad(table, ((0, 0), (0, 128 - DIM)))
    res = _lookup(table128, x_t, l, b)
    return jnp.transpose(res, (2, 0, 1))


# speedup vs baseline: 1.7051x; 1.4444x over previous
import functools

import jax
import jax.numpy as jnp
from jax import lax
from jax.experimental import pallas as pl
from jax.experimental.pallas import tpu as pltpu
from jax.experimental.pallas import tpu_sc as plsc

DIM = 64
SCALE = 8.0

_NC = 2
_NS = 16
_LANES = 16
_NW = _NC * _NS

_C = 256


@functools.partial(jax.jit, static_argnums=(2, 3))
def _lookup(table128, x_t, l_seq, b_total):
    stripe = b_total // _NW
    n_chunk = l_seq * (stripe // _C)
    mesh = plsc.VectorSubcoreMesh(core_axis_name="c", subcore_axis_name="s")

    @functools.partial(
        pl.kernel,
        mesh=mesh,
        out_type=jax.ShapeDtypeStruct((l_seq, DIM, b_total), jnp.float32),
        scratch_types=[
            [pltpu.VMEM((_C,), jnp.int32) for _ in range(2)],
            [pltpu.VMEM((_C, 128), jnp.float32) for _ in range(2)],
            [pltpu.VMEM((DIM, _C), jnp.float32) for _ in range(2)],
            [pltpu.SemaphoreType.DMA for _ in range(2)],
            [pltpu.SemaphoreType.DMA for _ in range(2)],
        ],
        compiler_params=pltpu.CompilerParams(needs_layout_passes=False),
    )
    def k(tab_hbm, xt_hbm, out_hbm, idx_v, pairs_v, tv_v, gsem, ssem):
        wid = lax.axis_index("s") * _NC + lax.axis_index("c")
        b0w = wid * stripe
        per_l = stripe // _C

        def chunk_lb(t):
            l = t // per_l
            b0 = b0w + (t % per_l) * _C
            return l, b0

        def start_gather(t, buf):
            l, b0 = chunk_lb(t)
            pltpu.sync_copy(xt_hbm.at[l, pl.ds(b0, _C)], idx_v[buf])
            return pltpu.async_copy(tab_hbm.at[idx_v[buf]], pairs_v[buf], gsem[buf])

        start_gather(0, 0)

        iota16 = lax.iota(jnp.int32, _LANES)

        def outer(tt, carry):
            for b in (0, 1):
                t = 2 * tt + b
                nxt = 1 - b

                if b == 0:
                    start_gather(t + 1, nxt)
                else:
                    @pl.when(tt < (n_chunk // 2 - 1))
                    def _():
                        start_gather(t + 1, nxt)

                pltpu.make_async_copy(
                    tab_hbm.at[idx_v[b]], pairs_v[b], gsem[b]
                ).wait()

                @pl.when(tt >= 1)
                def _():
                    l2, b02 = chunk_lb(t - 2)
                    pltpu.make_async_copy(
                        tv_v[b], out_hbm.at[l2, :, pl.ds(b02, _C)], ssem[b]
                    ).wait()

                @plsc.parallel_loop(0, _C, step=1, unroll=8)
                def _(i):
                    i_splat = jnp.full((_LANES,), 0, jnp.int32) + i
                    for kk in range(DIM // _LANES):
                        vals = pairs_v[b][i, pl.ds(kk * _LANES, _LANES)]
                        plsc.store_scatter(
                            tv_v[b],
                            [kk * _LANES + iota16, i_splat],
                            vals * SCALE,
                        )

                l, b0 = chunk_lb(t)
                pltpu.async_copy(
                    tv_v[b], out_hbm.at[l, :, pl.ds(b0, _C)], ssem[b]
                )
            return carry

        lax.fori_loop(0, n_chunk // 2, outer, 0)

        for b in (0, 1):
            t = n_chunk - 2 + b
            l2, b02 = chunk_lb(t)
            pltpu.make_async_copy(
                tv_v[b], out_hbm.at[l2, :, pl.ds(b02, _C)], ssem[b]
            ).wait()

    return k(table128, x_t)


def kernel(x, table):
    b, l = x.shape
    x_t = jnp.transpose(x).astype(jnp.int32)
    table128 = jnp.pad(table, ((0, 0), (0, 128 - DIM)))
    res = _lookup(table128, x_t, l, b)
    return jnp.transpose(res, (2, 0, 1))


# speedup vs baseline: 1.8385x; 1.0782x over previous
import functools

import jax
import jax.numpy as jnp
from jax import lax
from jax.experimental import pallas as pl
from jax.experimental.pallas import tpu as pltpu
from jax.experimental.pallas import tpu_sc as plsc

DIM = 64
SCALE = 8.0

_NC = 2
_NS = 16
_LANES = 16
_NW = _NC * _NS

_C = 128
_NB = 4


@functools.partial(jax.jit, static_argnums=(2, 3))
def _lookup(table128, x_t, l_seq, b_total):
    stripe = b_total // _NW
    per_l = stripe // _C
    assert per_l == _NB
    mesh = plsc.VectorSubcoreMesh(core_axis_name="c", subcore_axis_name="s")

    @functools.partial(
        pl.kernel,
        mesh=mesh,
        out_type=jax.ShapeDtypeStruct((l_seq, DIM, b_total), jnp.float32),
        scratch_types=[
            [pltpu.VMEM((_C,), jnp.int32) for _ in range(_NB)],
            [pltpu.VMEM((_C, 128), jnp.float32) for _ in range(_NB)],
            [pltpu.VMEM((DIM, _C), jnp.float32) for _ in range(2)],
            [pltpu.SemaphoreType.DMA for _ in range(_NB)],
            [pltpu.SemaphoreType.DMA for _ in range(_NB)],
            [pltpu.SemaphoreType.DMA for _ in range(2)],
        ],
        compiler_params=pltpu.CompilerParams(needs_layout_passes=False),
    )
    def k(tab_hbm, xt_hbm, out_hbm, idx_v, pairs_v, tv_v, isem, gsem, ssem):
        wid = lax.axis_index("s") * _NC + lax.axis_index("c")
        b0w = wid * stripe
        iota16 = lax.iota(jnp.int32, _LANES)

        def start_idx(l, slot):
            pltpu.async_copy(
                xt_hbm.at[l, pl.ds(b0w + slot * _C, _C)], idx_v[slot], isem[slot]
            )

        def wait_idx(slot):
            pltpu.make_async_copy(
                xt_hbm.at[0, pl.ds(b0w + slot * _C, _C)], idx_v[slot], isem[slot]
            ).wait()

        def start_gather(slot):
            pltpu.async_copy(tab_hbm.at[idx_v[slot]], pairs_v[slot], gsem[slot])

        def wait_gather(slot):
            pltpu.make_async_copy(
                tab_hbm.at[idx_v[slot]], pairs_v[slot], gsem[slot]
            ).wait()

        def start_store(l, slot):
            pltpu.async_copy(
                tv_v[slot % 2],
                out_hbm.at[l, :, pl.ds(b0w + slot * _C, _C)],
                ssem[slot % 2],
            )

        def wait_store(l, slot):
            pltpu.make_async_copy(
                tv_v[slot % 2],
                out_hbm.at[l, :, pl.ds(b0w + slot * _C, _C)],
                ssem[slot % 2],
            ).wait()

        for j in range(_NB):
            start_idx(0, j)
        for j in range(_NB - 1):
            wait_idx(j)
            start_gather(j)

        def outer(l, carry):
            for b in range(_NB):
                wait_gather(b)

                @pl.when(l < l_seq - 1)
                def _():
                    start_idx(l + 1, b)

                nslot = (b + 3) % _NB
                if b == 0:
                    wait_idx(nslot)
                    start_gather(nslot)
                else:
                    @pl.when(l < l_seq - 1)
                    def _():
                        wait_idx(nslot)
                        start_gather(nslot)

                if b >= 2:
                    wait_store(l, b - 2)
                else:
                    @pl.when(l >= 1)
                    def _():
                        wait_store(l - 1, b + 2)

                @plsc.parallel_loop(0, _C, step=1, unroll=8)
                def _(i):
                    i_splat = jnp.full((_LANES,), 0, jnp.int32) + i
                    for kk in range(DIM // _LANES):
                        vals = pairs_v[b][i, pl.ds(kk * _LANES, _LANES)]
                        plsc.store_scatter(
                            tv_v[b % 2],
                            [kk * _LANES + iota16, i_splat],
                            vals * SCALE,
                        )

                start_store(l, b)
            return carry

        lax.fori_loop(0, l_seq, outer, 0)

        wait_store(l_seq - 1, 2)
        wait_store(l_seq - 1, 3)

    return k(table128, x_t)


def kernel(x, table):
    b, l = x.shape
    x_t = jnp.transpose(x).astype(jnp.int32)
    table128 = jnp.pad(table, ((0, 0), (0, 128 - DIM)))
    res = _lookup(table128, x_t, l, b)
    return jnp.transpose(res, (2, 0, 1))
